# Initial kernel scaffold; baseline (speedup 1.0000x reference)
import jax
import jax.numpy as jnp
from jax.experimental import pallas as pl


def kernel(x, pe):
    raise NotImplementedError("write your pallas kernel here")


# speedup vs baseline: 3.4456x; 3.4456x over previous
import jax
import jax.numpy as jnp
from jax.experimental import pallas as pl

_SEQ_BLK = 512


def _pe_add_kernel(x_ref, pe_ref, o_ref):
    o_ref[...] = x_ref[...] + pe_ref[...] * 0.002


def kernel(x, pe):
    bz, lens, d = x.shape
    pe2 = pe[:lens, 0, :]
    grid = (bz, lens // _SEQ_BLK)
    return pl.pallas_call(
        _pe_add_kernel,
        grid=grid,
        in_specs=[
            pl.BlockSpec((1, _SEQ_BLK, d), lambda b, s: (b, s, 0)),
            pl.BlockSpec((_SEQ_BLK, d), lambda b, s: (s, 0)),
        ],
        out_specs=pl.BlockSpec((1, _SEQ_BLK, d), lambda b, s: (b, s, 0)),
        out_shape=jax.ShapeDtypeStruct((bz, lens, d), x.dtype),
    )(x, pe2)


# speedup vs baseline: 3.7183x; 1.0791x over previous
import jax
import jax.numpy as jnp
from jax.experimental import pallas as pl

_SEQ_BLK = 512


def _pe_add_kernel(x_ref, pe_ref, o_ref):
    o_ref[...] = x_ref[...] + pe_ref[...] * 0.002


def kernel(x, pe):
    bz, lens, d = x.shape
    pe2 = pe[:lens, 0, :]
    grid = (lens // _SEQ_BLK, bz)
    return pl.pallas_call(
        _pe_add_kernel,
        grid=grid,
        in_specs=[
            pl.BlockSpec((1, _SEQ_BLK, d), lambda s, b: (b, s, 0)),
            pl.BlockSpec((_SEQ_BLK, d), lambda s, b: (s, 0)),
        ],
        out_specs=pl.BlockSpec((1, _SEQ_BLK, d), lambda s, b: (b, s, 0)),
        out_shape=jax.ShapeDtypeStruct((bz, lens, d), x.dtype),
    )(x, pe2)


# speedup vs baseline: 3.9436x; 1.0606x over previous
import jax
import jax.numpy as jnp
from jax.experimental import pallas as pl

_SEQ_BLK = 512


def _pe_add_kernel(x_ref, pe_ref, o_ref):
    o_ref[...] = x_ref[...] + pe_ref[...][None, :, :] * 0.002


def kernel(x, pe):
    bz, lens, d = x.shape
    pe2 = pe[:lens, 0, :]
    grid = (lens // _SEQ_BLK,)
    return pl.pallas_call(
        _pe_add_kernel,
        grid=grid,
        in_specs=[
            pl.BlockSpec((bz, _SEQ_BLK, d), lambda s: (0, s, 0)),
            pl.BlockSpec((_SEQ_BLK, d), lambda s: (s, 0)),
        ],
        out_specs=pl.BlockSpec((bz, _SEQ_BLK, d), lambda s: (0, s, 0)),
        out_shape=jax.ShapeDtypeStruct((bz, lens, d), x.dtype),
    )(x, pe2)
